# baseline (speedup 1.0000x reference)
import jax
import jax.numpy as jnp
from jax import lax
from jax.experimental import pallas as pl
from jax.experimental.pallas import tpu as pltpu
from jax.experimental.pallas import tpu_sc as plsc

N_SUBCORES = 16
CHUNK = 128


def _acc_rows(N):
  return -(-N // (N_SUBCORES * 8)) * N_SUBCORES * 8


def _make_prop(N, D, chunks, e_pad):
  n_acc = _acc_rows(N)
  rows_per_tile = n_acc // N_SUBCORES
  f32 = jnp.float32
  mesh = plsc.VectorSubcoreMesh(core_axis_name="c", subcore_axis_name="s")

  scratch = [
      pltpu.VMEM((2, CHUNK), jnp.int32),
      pltpu.VMEM((2, CHUNK), jnp.int32),
      pltpu.VMEM((2 * CHUNK, D), f32),
      pltpu.VMEM_SHARED((n_acc + 8, D), f32),
      [pltpu.SemaphoreType.DMA for _ in range(2)],
      [pltpu.SemaphoreType.DMA for _ in range(2)],
      pltpu.VMEM((2, CHUNK), f32),
      [pltpu.SemaphoreType.DMA for _ in range(2)],
      [pltpu.SemaphoreType.DMA for _ in range(2)],
      pltpu.VMEM((16,), jnp.int32),
  ]
  tile_e = chunks * CHUNK

  def body(x0, x1, g0, d0, n0, g1, d1, n1, stepv, zeros,
           out0, out1, sidx, didx, rows, acc, ssem, gsem, nrmb, nsem,
           scsem, stepb):
    c = lax.axis_index("c")
    s = lax.axis_index("s")
    row0 = s * rows_per_tile
    pltpu.sync_copy(stepv, stepb)
    step = stepb[pl.ds(0, 16)][0]

    def run_graph(x, sh, dh, nhh, out):
      base = step * e_pad + s * tile_e
      pltpu.sync_copy(zeros, acc.at[pl.ds(row0, rows_per_tile)])

      def sld(k, b):
        return pltpu.make_async_copy(sh.at[pl.ds(base + k * CHUNK, CHUNK)],
                                     sidx.at[b], ssem[b])

      def gat(b):
        return pltpu.make_async_copy(x.at[sidx.at[b]],
                                     rows.at[pl.ds(b * CHUNK, CHUNK)],
                                     gsem[b])

      def dn(k, b):
        return (
            pltpu.make_async_copy(dh.at[pl.ds(base + k * CHUNK, CHUNK)],
                                  didx.at[b], nsem[b]),
            pltpu.make_async_copy(nhh.at[pl.ds(base + k * CHUNK, CHUNK)],
                                  nrmb.at[b], nsem[b]),
        )

      def scat(b):
        return pltpu.make_async_copy(rows.at[pl.ds(b * CHUNK, CHUNK)],
                                     acc.at[didx.at[b]], scsem[b])

      plsc.subcore_barrier()
      sld(0, 0).start()
      sld(1, 1).start()
      sld(0, 0).wait()
      gat(0).start()
      for d in dn(0, 0):
        d.start()

      def pair_body(i, carry):
        for par in range(2):
          k = 2 * i + par
          nxt = 1 - par

          @pl.when(k < chunks)
          def _():
            @pl.when(k + 1 < chunks)
            def _():
              sld(k + 1, nxt).wait()

              @pl.when(k >= 1)
              def _():
                scat(nxt).wait()

              gat(nxt).start()
              for d in dn(k + 1, nxt):
                d.start()

            gat(par).wait()

            @pl.when(k + 2 < chunks)
            def _():
              sld(k + 2, par).start()

            for d in dn(k, par):
              d.wait()

            def scale(e16, cc):
              nv16 = nrmb[par, pl.ds(e16 * 16, 16)]
              for l in range(16):
                nvec = jnp.full((16,), nv16[l], f32)
                e = e16 * 16 + l
                for j in range(D // 16):
                  sl = pl.ds(j * 16, 16)
                  rows[par * CHUNK + e, sl] = rows[par * CHUNK + e, sl] * nvec
              return cc

            lax.fori_loop(0, CHUNK // 16, scale, 0)
            scat(par).start(add=True)

        return carry

      lax.fori_loop(0, -(-chunks // 2), pair_body, 0)
      scat(0).wait()
      scat(1).wait()
      plsc.subcore_barrier()
      pltpu.sync_copy(acc.at[pl.ds(row0, rows_per_tile)],
                      out.at[pl.ds(row0, rows_per_tile)])

    @pl.when(c == 0)
    def _():
      run_graph(x0, g0, d0, n0, out0)

    @pl.when(c == 1)
    def _():
      run_graph(x1, g1, d1, n1, out1)

  out_type = [jax.ShapeDtypeStruct((n_acc, D), f32)] * 2
  return pl.kernel(body, out_type=out_type, mesh=mesh, scratch_types=scratch)


def _dense_layer(aq, at, Wq, Wt, mq, mt, flags):
  N, D = mq.shape
  R = 1000
  f32 = jnp.float32

  def body(flags_r, aq_r, at_r, wq_r, wt_r, mq_r, mt_r,
           xq_o, xt_o, mq_o, mt_o):
    elu = flags_r[0]
    cross = flags_r[1]
    xq = jnp.dot(aq_r[...], wq_r[...], preferred_element_type=f32)
    xt = jnp.dot(at_r[...], wt_r[...], preferred_element_type=f32)
    xq = jnp.where(elu > 0, jnp.where(xq > 0, xq, jnp.exp(xq) - 1.0), xq)
    xt = jnp.where(elu > 0, jnp.where(xt > 0, xt, jnp.exp(xt) - 1.0), xt)
    xq_o[...] = xq
    xt_o[...] = xt
    mq_o[...] = jnp.where(cross > 0, mq_r[...], jnp.maximum(mq_r[...], xq))
    mt_o[...] = jnp.where(cross > 0, mt_r[...], jnp.maximum(mt_r[...], xt))

  row = pl.BlockSpec((R, D), lambda i: (i, 0))
  w = pl.BlockSpec((D, D), lambda i: (0, 0))
  sm = pl.BlockSpec(memory_space=pltpu.SMEM)
  return pl.pallas_call(
      body,
      grid=(N // R,),
      in_specs=[sm, row, row, w, w, row, row],
      out_specs=[row, row, row, row],
      out_shape=[jax.ShapeDtypeStruct((N, D), f32)] * 4,
  )(flags, aq, at, Wq, Wt, mq, mt)


def _final_combine(Xq, Xt, cq, ct, Wiq, Wit, mask):
  N, D = Xq.shape
  R = 1000
  f32 = jnp.float32

  def body(xq_r, xt_r, cq_r, ct_r, wq_r, wt_r, m_r, oq, ot):
    oq[...] = xq_r[...] + m_r[...] * jnp.dot(
        cq_r[...], wq_r[...], preferred_element_type=f32)
    ot[...] = xt_r[...] + jnp.dot(
        ct_r[...], wt_r[...], preferred_element_type=f32)

  row = pl.BlockSpec((R, D), lambda i: (i, 0))
  w = pl.BlockSpec((D, D), lambda i: (0, 0))
  m = pl.BlockSpec((R, 1), lambda i: (i, 0))
  return pl.pallas_call(
      body,
      grid=(N // R,),
      in_specs=[row, row, row, row, w, w, m],
      out_specs=[row, row],
      out_shape=[jax.ShapeDtypeStruct((N, D), f32)] * 2,
  )(Xq, Xt, cq, ct, Wiq, Wit, mask)


def kernel(xq, xt, edge_index_q, edge_index_t, norm_q, norm_t, u2v, node_mask,
           Wq0, Wq1, Wq2, Wt0, Wt1, Wt2, Wint_q, Wint_t):
  N, D = xq.shape
  E = edge_index_q.shape[1]
  chunks = -(-(E // N_SUBCORES) // CHUNK)
  chunks = -(-chunks // 8) * 8
  e_pad = N_SUBCORES * chunks * CHUNK
  f32 = jnp.float32
  n_acc = _acc_rows(N)

  def prep(gather_idx, scatter_idx, nrm):
    pad = e_pad - E
    g = jnp.pad(gather_idx, (0, pad))
    sc = jnp.pad(scatter_idx, (0, pad), constant_values=N)
    return g, sc, jnp.pad(nrm, (0, pad))

  sq, dq, nq = prep(edge_index_q[0], edge_index_q[1], norm_q)
  st, dt, nt = prep(edge_index_t[0], edge_index_t[1], norm_t)
  ones = jnp.ones((E,), f32)
  gv, su, n1s = prep(u2v[1], u2v[0], ones)
  gu, sv, _ = prep(u2v[0], u2v[1], ones)

  zeros = jnp.zeros((n_acc // N_SUBCORES, D), f32)
  prop = _make_prop(N, D, chunks, e_pad)

  g0a = jnp.concatenate([sq, sq, sq, gv])
  d0a = jnp.concatenate([dq, dq, dq, su])
  n0a = jnp.concatenate([nq, nq, nq, n1s])
  g1a = jnp.concatenate([st, st, st, gu])
  d1a = jnp.concatenate([dt, dt, dt, sv])
  n1a = jnp.concatenate([nt, nt, nt, n1s])
  Wqs = jnp.stack([Wq0, Wq1, Wq2, Wq2])
  Wts = jnp.stack([Wt0, Wt1, Wt2, Wt2])
  flags = jnp.array([[1, 0], [1, 0], [0, 0], [0, 1]], jnp.int32)
  stepvs = jnp.broadcast_to(jnp.arange(4, dtype=jnp.int32)[:, None], (4, 16))

  def step(carry, xs):
    x_q, x_t, mq, mt, cq, ct = carry
    Wq, Wt, fl, sv16 = xs
    cross = fl[1] > 0
    xg0 = jnp.where(cross, mt, x_q)
    xg1 = jnp.where(cross, mq, x_t)
    a0, a1 = prop(xg0, xg1, g0a, d0a, n0a, g1a, d1a, n1a, sv16, zeros)
    x_qn, x_tn, mqn, mtn = _dense_layer(a0, a1, Wq, Wt, mq, mt, fl)
    cq = jnp.where(cross, a0, cq)
    ct = jnp.where(cross, a1, ct)
    return (x_qn, x_tn, mqn, mtn, cq, ct), 0

  carry0 = (xq, xt, xq, xt,
            jnp.zeros((n_acc, D), f32), jnp.zeros((n_acc, D), f32))
  (x_q, x_t, mq, mt, cq, ct), _ = lax.scan(
      step, carry0, (Wqs, Wts, flags, stepvs))

  return _final_combine(mq, mt, cq, ct, Wint_q, Wint_t,
                        node_mask.reshape(N, 1))


# speedup vs baseline: 1.0691x; 1.0691x over previous
import jax
import jax.numpy as jnp
from jax import lax
from jax.experimental import pallas as pl
from jax.experimental.pallas import tpu as pltpu
from jax.experimental.pallas import tpu_sc as plsc

N_SUBCORES = 16
CHUNK = 128


def _acc_rows(N):
  return -(-N // (N_SUBCORES * 8)) * N_SUBCORES * 8


def _make_prop(N, D, chunks, e_pad):
  n_acc = _acc_rows(N)
  rows_per_tile = n_acc // N_SUBCORES
  f32 = jnp.float32
  mesh = plsc.VectorSubcoreMesh(core_axis_name="c", subcore_axis_name="s")

  scratch = [
      pltpu.VMEM((2, CHUNK), jnp.int32),
      pltpu.VMEM((2, CHUNK), jnp.int32),
      pltpu.VMEM((2 * CHUNK, D), f32),
      pltpu.VMEM_SHARED((n_acc + 8, D), f32),
      [pltpu.SemaphoreType.DMA for _ in range(2)],
      [pltpu.SemaphoreType.DMA for _ in range(2)],
      pltpu.VMEM((2, CHUNK), f32),
      [pltpu.SemaphoreType.DMA for _ in range(2)],
      pltpu.VMEM((16,), jnp.int32),
  ]
  tile_e = chunks * CHUNK

  def body(x0, x1, g0, d0, n0, g1, d1, n1, stepv, zeros,
           out0, out1, sidx, didx, rows, acc, ssem, gsem, nrmb, nsem, stepb):
    c = lax.axis_index("c")
    s = lax.axis_index("s")
    row0 = s * rows_per_tile
    pltpu.sync_copy(stepv, stepb)
    step = stepb[pl.ds(0, 16)][0]

    def run_graph(x, sh, dh, nhh, out):
      base = step * e_pad + s * tile_e
      pltpu.sync_copy(zeros, acc.at[pl.ds(row0, rows_per_tile)])

      def sld(k, b):
        return pltpu.make_async_copy(sh.at[pl.ds(base + k * CHUNK, CHUNK)],
                                     sidx.at[b], ssem[b])

      def gat(b):
        return pltpu.make_async_copy(x.at[sidx.at[b]],
                                     rows.at[pl.ds(b * CHUNK, CHUNK)],
                                     gsem[b])

      def dn(k, b):
        return (
            pltpu.make_async_copy(dh.at[pl.ds(base + k * CHUNK, CHUNK)],
                                  didx.at[b], nsem[b]),
            pltpu.make_async_copy(nhh.at[pl.ds(base + k * CHUNK, CHUNK)],
                                  nrmb.at[b], nsem[b]),
        )

      plsc.subcore_barrier()
      sld(0, 0).start()
      sld(1, 1).start()
      sld(0, 0).wait()
      gat(0).start()
      for d in dn(0, 0):
        d.start()

      def pair_body(i, carry):
        for par in range(2):
          k = 2 * i + par
          nxt = 1 - par

          @pl.when(k < chunks)
          def _():
            @pl.when(k + 1 < chunks)
            def _():
              sld(k + 1, nxt).wait()
              gat(nxt).start()
              for d in dn(k + 1, nxt):
                d.start()

            gat(par).wait()

            @pl.when(k + 2 < chunks)
            def _():
              sld(k + 2, par).start()

            for d in dn(k, par):
              d.wait()

            def scale(e16, cc):
              nv16 = nrmb[par, pl.ds(e16 * 16, 16)]
              for l in range(16):
                nvec = jnp.full((16,), nv16[l], f32)
                e = e16 * 16 + l
                for j in range(D // 16):
                  sl = pl.ds(j * 16, 16)
                  rows[par * CHUNK + e, sl] = rows[par * CHUNK + e, sl] * nvec
              return cc

            lax.fori_loop(0, CHUNK // 16, scale, 0)
            pltpu.sync_copy(rows.at[pl.ds(par * CHUNK, CHUNK)],
                            acc.at[didx.at[par]], add=True)

        return carry

      lax.fori_loop(0, -(-chunks // 2), pair_body, 0)
      plsc.subcore_barrier()
      pltpu.sync_copy(acc.at[pl.ds(row0, rows_per_tile)],
                      out.at[pl.ds(row0, rows_per_tile)])

    @pl.when(c == 0)
    def _():
      run_graph(x0, g0, d0, n0, out0)

    @pl.when(c == 1)
    def _():
      run_graph(x1, g1, d1, n1, out1)

  out_type = [jax.ShapeDtypeStruct((n_acc, D), f32)] * 2
  return pl.kernel(body, out_type=out_type, mesh=mesh, scratch_types=scratch)


def _dense_layer(aq, at, Wq, Wt, mq, mt, flags):
  N, D = mq.shape
  R = 1000
  f32 = jnp.float32

  def body(flags_r, aq_r, at_r, wq_r, wt_r, mq_r, mt_r,
           xq_o, xt_o, mq_o, mt_o):
    elu = flags_r[0]
    cross = flags_r[1]
    xq = jnp.dot(aq_r[...], wq_r[...], preferred_element_type=f32)
    xt = jnp.dot(at_r[...], wt_r[...], preferred_element_type=f32)
    xq = jnp.where(elu > 0, jnp.where(xq > 0, xq, jnp.exp(xq) - 1.0), xq)
    xt = jnp.where(elu > 0, jnp.where(xt > 0, xt, jnp.exp(xt) - 1.0), xt)
    xq_o[...] = xq
    xt_o[...] = xt
    mq_o[...] = jnp.where(cross > 0, mq_r[...], jnp.maximum(mq_r[...], xq))
    mt_o[...] = jnp.where(cross > 0, mt_r[...], jnp.maximum(mt_r[...], xt))

  row = pl.BlockSpec((R, D), lambda i: (i, 0))
  w = pl.BlockSpec((D, D), lambda i: (0, 0))
  sm = pl.BlockSpec(memory_space=pltpu.SMEM)
  return pl.pallas_call(
      body,
      grid=(N // R,),
      in_specs=[sm, row, row, w, w, row, row],
      out_specs=[row, row, row, row],
      out_shape=[jax.ShapeDtypeStruct((N, D), f32)] * 4,
  )(flags, aq, at, Wq, Wt, mq, mt)


def _final_combine(Xq, Xt, cq, ct, Wiq, Wit, mask):
  N, D = Xq.shape
  R = 1000
  f32 = jnp.float32

  def body(xq_r, xt_r, cq_r, ct_r, wq_r, wt_r, m_r, oq, ot):
    oq[...] = xq_r[...] + m_r[...] * jnp.dot(
        cq_r[...], wq_r[...], preferred_element_type=f32)
    ot[...] = xt_r[...] + jnp.dot(
        ct_r[...], wt_r[...], preferred_element_type=f32)

  row = pl.BlockSpec((R, D), lambda i: (i, 0))
  w = pl.BlockSpec((D, D), lambda i: (0, 0))
  m = pl.BlockSpec((R, 1), lambda i: (i, 0))
  return pl.pallas_call(
      body,
      grid=(N // R,),
      in_specs=[row, row, row, row, w, w, m],
      out_specs=[row, row],
      out_shape=[jax.ShapeDtypeStruct((N, D), f32)] * 2,
  )(Xq, Xt, cq, ct, Wiq, Wit, mask)


def kernel(xq, xt, edge_index_q, edge_index_t, norm_q, norm_t, u2v, node_mask,
           Wq0, Wq1, Wq2, Wt0, Wt1, Wt2, Wint_q, Wint_t):
  N, D = xq.shape
  E = edge_index_q.shape[1]
  chunks = -(-(E // N_SUBCORES) // CHUNK)
  chunks = -(-chunks // 8) * 8
  e_pad = N_SUBCORES * chunks * CHUNK
  f32 = jnp.float32
  n_acc = _acc_rows(N)

  def prep(gather_idx, scatter_idx, nrm):
    pad = e_pad - E
    g = jnp.pad(gather_idx, (0, pad))
    sc = jnp.pad(scatter_idx, (0, pad), constant_values=N)
    return g, sc, jnp.pad(nrm, (0, pad))

  sq, dq, nq = prep(edge_index_q[0], edge_index_q[1], norm_q)
  st, dt, nt = prep(edge_index_t[0], edge_index_t[1], norm_t)
  ones = jnp.ones((E,), f32)
  gv, su, n1s = prep(u2v[1], u2v[0], ones)
  gu, sv, _ = prep(u2v[0], u2v[1], ones)

  zeros = jnp.zeros((n_acc // N_SUBCORES, D), f32)
  prop = _make_prop(N, D, chunks, e_pad)

  g0a = jnp.concatenate([sq, sq, sq, gv])
  d0a = jnp.concatenate([dq, dq, dq, su])
  n0a = jnp.concatenate([nq, nq, nq, n1s])
  g1a = jnp.concatenate([st, st, st, gu])
  d1a = jnp.concatenate([dt, dt, dt, sv])
  n1a = jnp.concatenate([nt, nt, nt, n1s])
  Wqs = jnp.stack([Wq0, Wq1, Wq2, Wq2])
  Wts = jnp.stack([Wt0, Wt1, Wt2, Wt2])
  flags = jnp.array([[1, 0], [1, 0], [0, 0], [0, 1]], jnp.int32)
  stepvs = jnp.broadcast_to(jnp.arange(4, dtype=jnp.int32)[:, None], (4, 16))

  def step(carry, xs):
    x_q, x_t, mq, mt, cq, ct = carry
    Wq, Wt, fl, sv16 = xs
    cross = fl[1] > 0
    xg0 = jnp.where(cross, mt, x_q)
    xg1 = jnp.where(cross, mq, x_t)
    a0, a1 = prop(xg0, xg1, g0a, d0a, n0a, g1a, d1a, n1a, sv16, zeros)
    x_qn, x_tn, mqn, mtn = _dense_layer(a0, a1, Wq, Wt, mq, mt, fl)
    cq = jnp.where(cross, a0, cq)
    ct = jnp.where(cross, a1, ct)
    return (x_qn, x_tn, mqn, mtn, cq, ct), 0

  carry0 = (xq, xt, xq, xt,
            jnp.zeros((n_acc, D), f32), jnp.zeros((n_acc, D), f32))
  (x_q, x_t, mq, mt, cq, ct), _ = lax.scan(
      step, carry0, (Wqs, Wts, flags, stepvs))

  return _final_combine(mq, mt, cq, ct, Wint_q, Wint_t,
                        node_mask.reshape(N, 1))
